# Initial kernel scaffold; baseline (speedup 1.0000x reference)
import jax
import jax.numpy as jnp
from jax.experimental import pallas as pl


def kernel(past_lengths, past_ids, past_embeddings, timestamps, ratings, rating_emb_weight):
    raise NotImplementedError("write your pallas kernel here")


# speedup vs baseline: 1.6152x; 1.6152x over previous
import jax
import jax.numpy as jnp
from jax.experimental import pallas as pl
from jax.experimental.pallas import tpu as pltpu

B, S, D = 4096, 200, 64
NUM_ROWS = 12
BB = 32


def _body(ratings_ref, past_ref, table_ref, out_ref):
    past = past_ref[...]
    r = ratings_ref[...]
    emb = jnp.zeros_like(past)
    for row in range(NUM_ROWS):
        mask = (r == row).astype(jnp.float32)[..., None]
        emb = emb + mask * table_ref[row, :]
    out_ref[...] = jnp.concatenate([past, emb], axis=-1)


def kernel(past_lengths, past_ids, past_embeddings, timestamps, ratings, rating_emb_weight):
    grid = (B // BB,)
    out = pl.pallas_call(
        _body,
        grid=grid,
        in_specs=[
            pl.BlockSpec((BB, S), lambda i: (i, 0)),
            pl.BlockSpec((BB, S, D), lambda i: (i, 0, 0)),
            pl.BlockSpec((NUM_ROWS, D), lambda i: (0, 0)),
        ],
        out_specs=pl.BlockSpec((BB, S, 2 * D), lambda i: (i, 0, 0)),
        out_shape=jax.ShapeDtypeStruct((B, S, 2 * D), jnp.float32),
        compiler_params=pltpu.CompilerParams(
            dimension_semantics=("arbitrary",),
        ),
    )(ratings, past_embeddings, rating_emb_weight)
    return out.reshape(B, 2 * S, D)


# speedup vs baseline: 2.4838x; 1.5378x over previous
import jax
import jax.numpy as jnp
from jax.experimental import pallas as pl
from jax.experimental.pallas import tpu as pltpu

B, S, D = 4096, 200, 64
NUM_ROWS = 12
GRID = 128
ROWS = (B * S) // GRID


def _body(ratings_ref, past_ref, table_ref, out_ref):
    r = ratings_ref[...].reshape(1, ROWS)
    iota = jax.lax.broadcasted_iota(jnp.int32, (NUM_ROWS, ROWS), 0)
    onehot_t = (iota == r).astype(jnp.float32)
    emb = jax.lax.dot_general(
        onehot_t, table_ref[...],
        (((0,), (0,)), ((), ())),
        preferred_element_type=jnp.float32,
    )
    out_ref[:, 0:D] = past_ref[...]
    out_ref[:, D:2 * D] = emb


def kernel(past_lengths, past_ids, past_embeddings, timestamps, ratings, rating_emb_weight):
    past2d = past_embeddings.reshape(B * S, D)
    ratings3d = ratings.reshape(GRID, 1, ROWS)
    out = pl.pallas_call(
        _body,
        grid=(GRID,),
        in_specs=[
            pl.BlockSpec((1, 1, ROWS), lambda i: (i, 0, 0)),
            pl.BlockSpec((ROWS, D), lambda i: (i, 0)),
            pl.BlockSpec((NUM_ROWS, D), lambda i: (0, 0)),
        ],
        out_specs=pl.BlockSpec((ROWS, 2 * D), lambda i: (i, 0)),
        out_shape=jax.ShapeDtypeStruct((B * S, 2 * D), jnp.float32),
        compiler_params=pltpu.CompilerParams(
            dimension_semantics=("arbitrary",),
        ),
    )(ratings3d, past2d, rating_emb_weight)
    return out.reshape(B, 2 * S, D)
